# Initial kernel scaffold; baseline (speedup 1.0000x reference)
import jax
import jax.numpy as jnp
from jax.experimental import pallas as pl
from jax.experimental.pallas import tpu as pltpu

H = 224
W = 224
T = 16
F = 8
C = 64
K = T * F
HALO = 4
R = 28
NB = H // R
NR = R + 2 * HALO
P = NR * W


def _shift(a, k):
    if k > 0:
        return jnp.concatenate([a[-k:], a[:-k]], axis=0)
    return jnp.concatenate([a[-k:], a[:k]], axis=0)


def _body(x_hbm, w0, b0, wr0, br0, wr1, br1, w4t, b4, out_ref, xblk, sem):
    i = pl.program_id(0)
    base = i * R * W
    cp = pltpu.make_async_copy(x_hbm.at[pl.ds(base, P), :], xblk, sem)
    cp.start()
    cp.wait()

    def geom(width):
        pio = jax.lax.broadcasted_iota(jnp.int32, (P, width), 0) + base
        row = pio // W - HALO
        col = pio % W
        vp = 1 + (row > 0).astype(jnp.int32) + (row < H - 1).astype(jnp.int32)
        hp = 1 + (col > 0).astype(jnp.int32) + (col < W - 1).astype(jnp.int32)
        dinv = jax.lax.rsqrt((vp * hp + 1).astype(jnp.float32))
        in_img = jnp.logical_and(row >= 0, row <= H - 1)
        dinv = jnp.where(in_img, dinv, 0.0)
        maskl = col > 0
        maskr = col < W - 1
        return dinv, maskl, maskr

    dinv, maskl, maskr = geom(C)

    def agg(xw, dinv_, maskl_, maskr_, brow):
        s = dinv_ * xw
        csum = s \
            + jnp.where(maskl_, _shift(s, 1), 0.0) \
            + jnp.where(maskr_, _shift(s, -1), 0.0)
        box = csum + _shift(csum, W) + _shift(csum, -W)
        return dinv_ * (box + s) + brow

    def gcn(inp, w, brow):
        xw = jnp.dot(inp, w, preferred_element_type=jnp.float32)
        return agg(xw, dinv, maskl, maskr, brow)

    x = xblk[...]
    h0 = gcn(x, w0[...], b0[...])
    h1 = gcn(jax.nn.relu(h0), wr0[...], br0[...])
    h2 = gcn(jax.nn.relu(h1), wr1[...], br1[...])
    ls = jax.nn.relu(h0 + h1 + h2)

    xw4 = jnp.sum(ls * w4t[...], axis=1, keepdims=True)
    dinv1, maskl1, maskr1 = geom(1)
    out = agg(xw4, dinv1, maskl1, maskr1, b4[...])
    out_ref[...] = out[HALO * W:(HALO + R) * W, :]


def kernel(l_input, y, W0, b0, Wr0, br0, Wr1, br1, W4, b4):
    n = H * W
    x = jnp.transpose(l_input, (0, 2, 3, 4, 1)).reshape(n, K)
    zpad = jnp.zeros((HALO * W, K), jnp.float32)
    x_pad = jnp.concatenate([zpad, x, zpad], axis=0)

    out = pl.pallas_call(
        _body,
        grid=(NB,),
        in_specs=[
            pl.BlockSpec(memory_space=pltpu.MemorySpace.HBM),
            pl.BlockSpec((K, C), lambda i: (0, 0)),
            pl.BlockSpec((1, C), lambda i: (0, 0)),
            pl.BlockSpec((C, C), lambda i: (0, 0)),
            pl.BlockSpec((1, C), lambda i: (0, 0)),
            pl.BlockSpec((C, C), lambda i: (0, 0)),
            pl.BlockSpec((1, C), lambda i: (0, 0)),
            pl.BlockSpec((1, C), lambda i: (0, 0)),
            pl.BlockSpec((1, 1), lambda i: (0, 0)),
        ],
        out_specs=pl.BlockSpec((R * W, 1), lambda i: (i, 0)),
        out_shape=jax.ShapeDtypeStruct((n, 1), jnp.float32),
        scratch_shapes=[
            pltpu.VMEM((P, K), jnp.float32),
            pltpu.SemaphoreType.DMA,
        ],
    )(x_pad, W0, b0.reshape(1, C), Wr0, br0.reshape(1, C),
      Wr1, br1.reshape(1, C), W4.reshape(1, C), b4.reshape(1, 1))

    return out.reshape(1, 1, H, W, 1)


# speedup vs baseline: 56.0211x; 56.0211x over previous
import jax
import jax.numpy as jnp
from jax.experimental import pallas as pl
from jax.experimental.pallas import tpu as pltpu

H = 224
W = 224
T = 16
F = 8
C = 64
K = T * F
HALO = 4
R = 28
NB = H // R
NR = R + 2 * HALO
P = NR * W


def _shift(a, k):
    return jnp.concatenate([a[-k:], a[:-k]], axis=0)


def _body(x_hbm, w0, b0, wr0, br0, wr1, br1, w4t, b4, out_ref, xblk, sem):
    i = pl.program_id(0)
    base = i * R * W
    cp = pltpu.make_async_copy(x_hbm.at[pl.ds(base, P), :], xblk, sem)
    cp.start()
    cp.wait()

    def geom(width):
        pio = jax.lax.broadcasted_iota(jnp.int32, (P, width), 0) + base
        row = pio // W - HALO
        col = pio % W
        vp = 1 + (row > 0).astype(jnp.int32) + (row < H - 1).astype(jnp.int32)
        hp = 1 + (col > 0).astype(jnp.int32) + (col < W - 1).astype(jnp.int32)
        dinv = jax.lax.rsqrt((vp * hp + 1).astype(jnp.float32))
        in_img = jnp.logical_and(row >= 0, row <= H - 1)
        dinv = jnp.where(in_img, dinv, 0.0)
        maskl = col > 0
        maskr = col < W - 1
        return dinv, maskl, maskr

    dinv, maskl, maskr = geom(C)

    def agg(xw, dinv_, maskl_, maskr_, brow):
        s = dinv_ * xw
        csum = s \
            + jnp.where(maskl_, _shift(s, 1), 0.0) \
            + jnp.where(maskr_, _shift(s, -1), 0.0)
        box = csum + _shift(csum, W) + _shift(csum, -W)
        return dinv_ * (box + s) + brow

    def gcn(inp, w, brow):
        xw = jnp.dot(inp, w, preferred_element_type=jnp.float32)
        return agg(xw, dinv, maskl, maskr, brow)

    x = xblk[...]
    h0 = gcn(x, w0[...], b0[...])
    h1 = gcn(jax.nn.relu(h0), wr0[...], br0[...])
    h2 = gcn(jax.nn.relu(h1), wr1[...], br1[...])
    ls = jax.nn.relu(h0 + h1 + h2)

    xw4 = jnp.sum(ls * w4t[...], axis=1, keepdims=True)
    dinv1, maskl1, maskr1 = geom(1)
    out = agg(xw4, dinv1, maskl1, maskr1, b4[...])
    out_ref[...] = out[HALO * W:(HALO + R) * W, :]


def kernel(l_input, y, W0, b0, Wr0, br0, Wr1, br1, W4, b4):
    n = H * W
    x = jnp.transpose(l_input, (0, 2, 3, 4, 1)).reshape(n, K)
    zpad = jnp.zeros((HALO * W, K), jnp.float32)
    x_pad = jnp.concatenate([zpad, x, zpad], axis=0)

    out = pl.pallas_call(
        _body,
        grid=(NB,),
        in_specs=[
            pl.BlockSpec(memory_space=pltpu.MemorySpace.HBM),
            pl.BlockSpec((K, C), lambda i: (0, 0)),
            pl.BlockSpec((1, C), lambda i: (0, 0)),
            pl.BlockSpec((C, C), lambda i: (0, 0)),
            pl.BlockSpec((1, C), lambda i: (0, 0)),
            pl.BlockSpec((C, C), lambda i: (0, 0)),
            pl.BlockSpec((1, C), lambda i: (0, 0)),
            pl.BlockSpec((1, C), lambda i: (0, 0)),
            pl.BlockSpec((1, 1), lambda i: (0, 0)),
        ],
        out_specs=pl.BlockSpec((R * W, 1), lambda i: (i, 0)),
        out_shape=jax.ShapeDtypeStruct((n, 1), jnp.float32),
        scratch_shapes=[
            pltpu.VMEM((P, K), jnp.float32),
            pltpu.SemaphoreType.DMA,
        ],
    )(x_pad, W0, b0.reshape(1, C), Wr0, br0.reshape(1, C),
      Wr1, br1.reshape(1, C), W4.reshape(1, C), b4.reshape(1, 1))

    return out.reshape(1, 1, H, W, 1)


# speedup vs baseline: 59.9576x; 1.0703x over previous
import jax
import jax.numpy as jnp
from jax.experimental import pallas as pl
from jax.experimental.pallas import tpu as pltpu

H = 224
W = 224
T = 16
F = 8
C = 64
K = T * F
HALO = 4
R = 56
NB = H // R
NR = R + 2 * HALO
P = NR * W


def _shift(a, k):
    return jnp.concatenate([a[-k:], a[:-k]], axis=0)


def _body(x_hbm, w0, b0, wr0, br0, wr1, br1, w4t, b4, out_ref, xblk, sem):
    i = pl.program_id(0)
    base = i * R * W
    cp = pltpu.make_async_copy(x_hbm.at[pl.ds(base, P), :], xblk, sem)
    cp.start()
    cp.wait()

    def geom(width):
        pio = jax.lax.broadcasted_iota(jnp.int32, (P, width), 0) + base
        row = pio // W - HALO
        col = pio % W
        vp = 1 + (row > 0).astype(jnp.int32) + (row < H - 1).astype(jnp.int32)
        hp = 1 + (col > 0).astype(jnp.int32) + (col < W - 1).astype(jnp.int32)
        dinv = jax.lax.rsqrt((vp * hp + 1).astype(jnp.float32))
        in_img = jnp.logical_and(row >= 0, row <= H - 1)
        dinv = jnp.where(in_img, dinv, 0.0)
        maskl = col > 0
        maskr = col < W - 1
        return dinv, maskl, maskr

    dinv, maskl, maskr = geom(C)

    def agg(xw, dinv_, maskl_, maskr_, brow):
        s = dinv_ * xw
        csum = s \
            + jnp.where(maskl_, _shift(s, 1), 0.0) \
            + jnp.where(maskr_, _shift(s, -1), 0.0)
        box = csum + _shift(csum, W) + _shift(csum, -W)
        return dinv_ * (box + s) + brow

    def gcn(inp, w, brow):
        xw = jnp.dot(inp, w, preferred_element_type=jnp.float32)
        return agg(xw, dinv, maskl, maskr, brow)

    x = xblk[...]
    h0 = gcn(x, w0[...], b0[...])
    h1 = gcn(jax.nn.relu(h0), wr0[...], br0[...])
    h2 = gcn(jax.nn.relu(h1), wr1[...], br1[...])
    ls = jax.nn.relu(h0 + h1 + h2)

    xw4 = jnp.sum(ls * w4t[...], axis=1, keepdims=True)
    dinv1, maskl1, maskr1 = geom(1)
    out = agg(xw4, dinv1, maskl1, maskr1, b4[...])
    out_ref[...] = out[HALO * W:(HALO + R) * W, :]


def kernel(l_input, y, W0, b0, Wr0, br0, Wr1, br1, W4, b4):
    n = H * W
    x = jnp.transpose(l_input, (0, 2, 3, 4, 1)).reshape(n, K)
    zpad = jnp.zeros((HALO * W, K), jnp.float32)
    x_pad = jnp.concatenate([zpad, x, zpad], axis=0)

    out = pl.pallas_call(
        _body,
        grid=(NB,),
        in_specs=[
            pl.BlockSpec(memory_space=pltpu.MemorySpace.HBM),
            pl.BlockSpec((K, C), lambda i: (0, 0)),
            pl.BlockSpec((1, C), lambda i: (0, 0)),
            pl.BlockSpec((C, C), lambda i: (0, 0)),
            pl.BlockSpec((1, C), lambda i: (0, 0)),
            pl.BlockSpec((C, C), lambda i: (0, 0)),
            pl.BlockSpec((1, C), lambda i: (0, 0)),
            pl.BlockSpec((1, C), lambda i: (0, 0)),
            pl.BlockSpec((1, 1), lambda i: (0, 0)),
        ],
        out_specs=pl.BlockSpec((R * W, 1), lambda i: (i, 0)),
        out_shape=jax.ShapeDtypeStruct((n, 1), jnp.float32),
        scratch_shapes=[
            pltpu.VMEM((P, K), jnp.float32),
            pltpu.SemaphoreType.DMA,
        ],
    )(x_pad, W0, b0.reshape(1, C), Wr0, br0.reshape(1, C),
      Wr1, br1.reshape(1, C), W4.reshape(1, C), b4.reshape(1, 1))

    return out.reshape(1, 1, H, W, 1)


# speedup vs baseline: 72.8343x; 1.2148x over previous
import numpy as np
import jax
import jax.numpy as jnp
from jax.experimental import pallas as pl
from jax.experimental.pallas import tpu as pltpu

H = 224
W = 224
T = 16
F = 8
C = 64
K = T * F
HALO = 4
HP = H + 2 * HALO
WP = W + 2 * HALO
R = 56
NB = H // R
NR = R + 2 * HALO
P = NR * WP


def _dinv_const():
    vp = np.zeros(HP, np.float64)
    r = np.arange(HP) - HALO
    inside = (r >= 0) & (r < H)
    vp[inside] = 1 + (r[inside] > 0) + (r[inside] < H - 1)
    hp = np.zeros(WP, np.float64)
    c = np.arange(WP) - HALO
    insc = (c >= 0) & (c < W)
    hp[insc] = 1 + (c[insc] > 0) + (c[insc] < W - 1)
    deg = vp[:, None] * hp[None, :] + 1.0
    dinv = np.where(inside[:, None] & insc[None, :], 1.0 / np.sqrt(deg), 0.0)
    return np.ascontiguousarray(
        np.broadcast_to(dinv.reshape(HP * WP, 1), (HP * WP, C))
    ).astype(np.float32)


_DINV = _dinv_const()


def _shift(a, k):
    return jnp.concatenate([a[-k:], a[:-k]], axis=0)


def _body(x_hbm, dv_hbm, w0, b0, wr0, br0, wr1, br1, w4w, b4, out_ref,
          xblk, dvblk, sem, sem2):
    i = pl.program_id(0)
    base = i * R * WP
    cp1 = pltpu.make_async_copy(x_hbm.at[pl.ds(base, P), :], xblk, sem)
    cp2 = pltpu.make_async_copy(dv_hbm.at[pl.ds(base, P), :], dvblk, sem2)
    cp1.start()
    cp2.start()
    cp1.wait()
    cp2.wait()

    dinv = dvblk[...]

    def agg(xw, brow):
        s = dinv * xw
        csum = s + _shift(s, 1) + _shift(s, -1)
        box = csum + _shift(csum, WP) + _shift(csum, -WP)
        return dinv * (box + s) + brow

    def gcn(inp, w, brow):
        return agg(jnp.dot(inp, w, preferred_element_type=jnp.float32), brow)

    x = xblk[...]
    h0 = gcn(x, w0[...], b0[...])
    h1 = gcn(jax.nn.relu(h0), wr0[...], br0[...])
    h2 = gcn(jax.nn.relu(h1), wr1[...], br1[...])
    ls = jax.nn.relu(h0 + h1 + h2)

    h4 = gcn(ls, w4w[...], b4[...])
    out_ref[...] = h4[HALO * WP:(HALO + R) * WP, :1]


def kernel(l_input, y, W0, b0, Wr0, br0, Wr1, br1, W4, b4):
    xp = jnp.pad(l_input, ((0, 0), (0, 0), (HALO, HALO), (HALO, HALO), (0, 0)))
    x_pad = jnp.transpose(xp, (0, 2, 3, 4, 1)).reshape(HP * WP, K)
    dinv = jnp.asarray(_DINV)
    w4w = jnp.tile(W4, (1, C))

    out = pl.pallas_call(
        _body,
        grid=(NB,),
        in_specs=[
            pl.BlockSpec(memory_space=pltpu.MemorySpace.HBM),
            pl.BlockSpec(memory_space=pltpu.MemorySpace.HBM),
            pl.BlockSpec((K, C), lambda i: (0, 0)),
            pl.BlockSpec((1, C), lambda i: (0, 0)),
            pl.BlockSpec((C, C), lambda i: (0, 0)),
            pl.BlockSpec((1, C), lambda i: (0, 0)),
            pl.BlockSpec((C, C), lambda i: (0, 0)),
            pl.BlockSpec((1, C), lambda i: (0, 0)),
            pl.BlockSpec((C, C), lambda i: (0, 0)),
            pl.BlockSpec((1, 1), lambda i: (0, 0)),
        ],
        out_specs=pl.BlockSpec((R * WP, 1), lambda i: (i, 0)),
        out_shape=jax.ShapeDtypeStruct((H * WP, 1), jnp.float32),
        scratch_shapes=[
            pltpu.VMEM((P, K), jnp.float32),
            pltpu.VMEM((P, C), jnp.float32),
            pltpu.SemaphoreType.DMA,
            pltpu.SemaphoreType.DMA,
        ],
    )(x_pad, dinv, W0, b0.reshape(1, C), Wr0, br0.reshape(1, C),
      Wr1, br1.reshape(1, C), w4w, b4.reshape(1, 1))

    out = out.reshape(H, WP)[:, HALO:HALO + W]
    return out.reshape(1, 1, H, W, 1)


# speedup vs baseline: 92.1375x; 1.2650x over previous
import numpy as np
import jax
import jax.numpy as jnp
from jax.experimental import pallas as pl
from jax.experimental.pallas import tpu as pltpu

H = 224
W = 224
T = 16
F = 8
C = 64
K = T * F
HALO = 4
HP = H + 2 * HALO
WP = W + 2 * HALO
R = 56
NB = H // R
NR = R + 2 * HALO
P = NR * WP


def _dinv_const():
    vp = np.zeros(HP, np.float64)
    r = np.arange(HP) - HALO
    inside = (r >= 0) & (r < H)
    vp[inside] = 1 + (r[inside] > 0) + (r[inside] < H - 1)
    hp = np.zeros(WP, np.float64)
    c = np.arange(WP) - HALO
    insc = (c >= 0) & (c < W)
    hp[insc] = 1 + (c[insc] > 0) + (c[insc] < W - 1)
    deg = vp[:, None] * hp[None, :] + 1.0
    dinv = np.where(inside[:, None] & insc[None, :], 1.0 / np.sqrt(deg), 0.0)
    return dinv.astype(np.float32)


_DINV2D = _dinv_const()
_DINV = np.ascontiguousarray(
    np.broadcast_to(_DINV2D.reshape(HP * WP, 1), (HP * WP, C))
).astype(np.float32)


def _shift(a, k):
    return jnp.concatenate([a[-k:], a[:-k]], axis=0)


def _body(x_hbm, dv_hbm, dv2d, w0, b0, wr0, br0, wr1, br1, w4, b4, out_ref,
          xblk, dvblk, sem_x, sem_d):
    i = pl.program_id(0)
    slot = jax.lax.rem(i, 2)

    def start(blk, s):
        base = blk * R * WP
        pltpu.make_async_copy(
            x_hbm.at[pl.ds(base, P), :], xblk.at[s], sem_x.at[s]).start()
        pltpu.make_async_copy(
            dv_hbm.at[pl.ds(base, P), :], dvblk.at[s], sem_d.at[s]).start()

    @pl.when(i == 0)
    def _():
        start(0, 0)

    @pl.when(i + 1 < NB)
    def _():
        start(i + 1, 1 - slot)

    pltpu.make_async_copy(
        x_hbm.at[pl.ds(i * R * WP, P), :], xblk.at[slot], sem_x.at[slot]).wait()
    pltpu.make_async_copy(
        dv_hbm.at[pl.ds(i * R * WP, P), :], dvblk.at[slot], sem_d.at[slot]).wait()

    dinv = dvblk[slot]

    def agg(xw, brow):
        s = dinv * xw
        csum = s + _shift(s, 1) + _shift(s, -1)
        box = csum + _shift(csum, WP) + _shift(csum, -WP)
        return dinv * (box + s) + brow

    def gcn(inp, w, brow):
        return agg(jnp.dot(inp, w, preferred_element_type=jnp.float32), brow)

    x = xblk[slot]
    h0 = gcn(x, w0[...], b0[...])
    h1 = gcn(jax.nn.relu(h0), wr0[...], br0[...])
    h2 = gcn(jax.nn.relu(h1), wr1[...], br1[...])
    ls = jax.nn.relu(h0 + h1 + h2)

    xw4 = jnp.dot(ls, w4[...], preferred_element_type=jnp.float32)
    f2 = xw4.reshape(NR, WP)
    dv2 = dv2d[pl.ds(i * R, NR), :]
    s3 = dv2 * f2
    c3 = s3 \
        + jnp.concatenate([s3[:, -1:], s3[:, :-1]], axis=1) \
        + jnp.concatenate([s3[:, 1:], s3[:, :1]], axis=1)
    b3 = c3 \
        + jnp.concatenate([c3[-1:], c3[:-1]], axis=0) \
        + jnp.concatenate([c3[1:], c3[:1]], axis=0)
    h4 = dv2 * (b3 + s3) + b4[...]
    out_ref[...] = h4[HALO:HALO + R, :]


def kernel(l_input, y, W0, b0, Wr0, br0, Wr1, br1, W4, b4):
    xp = jnp.pad(l_input, ((0, 0), (0, 0), (HALO, HALO), (HALO, HALO), (0, 0)))
    x_pad = jnp.transpose(xp, (0, 2, 3, 4, 1)).reshape(HP * WP, K)
    dinv = jnp.asarray(_DINV)
    dinv2d = jnp.asarray(_DINV2D)

    out = pl.pallas_call(
        _body,
        grid=(NB,),
        in_specs=[
            pl.BlockSpec(memory_space=pltpu.MemorySpace.HBM),
            pl.BlockSpec(memory_space=pltpu.MemorySpace.HBM),
            pl.BlockSpec((HP, WP), lambda i: (0, 0)),
            pl.BlockSpec((K, C), lambda i: (0, 0)),
            pl.BlockSpec((1, C), lambda i: (0, 0)),
            pl.BlockSpec((C, C), lambda i: (0, 0)),
            pl.BlockSpec((1, C), lambda i: (0, 0)),
            pl.BlockSpec((C, C), lambda i: (0, 0)),
            pl.BlockSpec((1, C), lambda i: (0, 0)),
            pl.BlockSpec((C, 1), lambda i: (0, 0)),
            pl.BlockSpec((1, 1), lambda i: (0, 0)),
        ],
        out_specs=pl.BlockSpec((R, WP), lambda i: (i, 0)),
        out_shape=jax.ShapeDtypeStruct((H, WP), jnp.float32),
        scratch_shapes=[
            pltpu.VMEM((2, P, K), jnp.float32),
            pltpu.VMEM((2, P, C), jnp.float32),
            pltpu.SemaphoreType.DMA((2,)),
            pltpu.SemaphoreType.DMA((2,)),
        ],
    )(x_pad, dinv, dinv2d, W0, b0.reshape(1, C), Wr0, br0.reshape(1, C),
      Wr1, br1.reshape(1, C), W4, b4.reshape(1, 1))

    out = out[:, HALO:HALO + W]
    return out.reshape(1, 1, H, W, 1)


# speedup vs baseline: 126.8290x; 1.3765x over previous
import numpy as np
import jax
import jax.numpy as jnp
from jax.experimental import pallas as pl
from jax.experimental.pallas import tpu as pltpu

H = 224
W = 224
T = 16
F = 8
C = 64
K = T * F
HALO = 4
HP = H + 2 * HALO
WP = W + 2 * HALO
R = 56
NB = H // R
NR = R + 2 * HALO
P = NR * WP


def _dinv_const():
    vp = np.zeros(HP, np.float64)
    r = np.arange(HP) - HALO
    inside = (r >= 0) & (r < H)
    vp[inside] = 1 + (r[inside] > 0) + (r[inside] < H - 1)
    hp = np.zeros(WP, np.float64)
    c = np.arange(WP) - HALO
    insc = (c >= 0) & (c < W)
    hp[insc] = 1 + (c[insc] > 0) + (c[insc] < W - 1)
    deg = vp[:, None] * hp[None, :] + 1.0
    dinv = np.where(inside[:, None] & insc[None, :], 1.0 / np.sqrt(deg), 0.0)
    return dinv.astype(np.float32)


_DINV2D = _dinv_const()
_DINV = np.ascontiguousarray(
    np.broadcast_to(_DINV2D.reshape(HP * WP, 1), (HP * WP, C))
).astype(np.float32)


def _shift(a, k):
    return jnp.concatenate([a[-k:], a[:-k]], axis=0)


def _body(x_hbm, dv_hbm, dv2d, w0, b0, wr0, br0, wr1, br1, w4, b4, out_ref,
          xblk, dvblk, sem_x, sem_d):
    i = pl.program_id(0)
    slot = jax.lax.rem(i, 2)

    def xcopy(blk, s, start):
        def mk(src_r0, dst_r0, nrows):
            return pltpu.make_async_copy(
                x_hbm.at[pl.ds(src_r0, nrows), :, :],
                xblk.at[s, pl.ds(dst_r0, nrows), pl.ds(HALO, W), :],
                sem_x.at[s])

        @pl.when(blk == 0)
        def _():
            cp = mk(0, HALO, NR - HALO)
            cp.start() if start else cp.wait()

        r0 = jnp.maximum(blk * R - HALO, 0)

        @pl.when(jnp.logical_and(blk > 0, blk < NB - 1))
        def _():
            cp = mk(r0, 0, NR)
            cp.start() if start else cp.wait()

        @pl.when(blk == NB - 1)
        def _():
            cp = mk(r0, 0, NR - HALO)
            cp.start() if start else cp.wait()

    def dcopy(blk, s, start):
        cp = pltpu.make_async_copy(
            dv_hbm.at[pl.ds(blk * R * WP, P), :], dvblk.at[s], sem_d.at[s])
        cp.start() if start else cp.wait()

    @pl.when(i == 0)
    def _():
        xblk[0, :, 0:HALO, :] = jnp.zeros((NR, HALO, K), jnp.float32)
        xblk[1, :, 0:HALO, :] = jnp.zeros((NR, HALO, K), jnp.float32)
        xblk[0, :, HALO + W:WP, :] = jnp.zeros((NR, HALO, K), jnp.float32)
        xblk[1, :, HALO + W:WP, :] = jnp.zeros((NR, HALO, K), jnp.float32)
        xblk[0, 0:HALO, pl.ds(HALO, W), :] = jnp.zeros((HALO, W, K), jnp.float32)
        xcopy(0, 0, True)
        dcopy(0, 0, True)

    @pl.when(i + 1 < NB)
    def _():
        xcopy(i + 1, 1 - slot, True)
        dcopy(i + 1, 1 - slot, True)

    xcopy(i, slot, False)
    dcopy(i, slot, False)

    dinv = dvblk[slot]

    def agg(xw, brow):
        s = dinv * xw
        csum = s + _shift(s, 1) + _shift(s, -1)
        box = csum + _shift(csum, WP) + _shift(csum, -WP)
        return dinv * (box + s) + brow

    def gcn(inp, w, brow):
        return agg(jnp.dot(inp, w, preferred_element_type=jnp.float32), brow)

    x = jnp.reshape(xblk[slot], (P, K))
    h0 = gcn(x, w0[...], b0[...])
    h1 = gcn(jax.nn.relu(h0), wr0[...], br0[...])
    h2 = gcn(jax.nn.relu(h1), wr1[...], br1[...])
    ls = jax.nn.relu(h0 + h1 + h2)

    xw4 = jnp.dot(ls, w4[...], preferred_element_type=jnp.float32)
    f2 = xw4.reshape(NR, WP)
    dv2 = dv2d[pl.ds(i * R, NR), :]
    s3 = dv2 * f2
    c3 = s3 \
        + jnp.concatenate([s3[:, -1:], s3[:, :-1]], axis=1) \
        + jnp.concatenate([s3[:, 1:], s3[:, :1]], axis=1)
    b3 = c3 \
        + jnp.concatenate([c3[-1:], c3[:-1]], axis=0) \
        + jnp.concatenate([c3[1:], c3[:1]], axis=0)
    h4 = dv2 * (b3 + s3) + b4[...]
    out_ref[...] = h4[HALO:HALO + R, :]


def kernel(l_input, y, W0, b0, Wr0, br0, Wr1, br1, W4, b4):
    x3 = jnp.transpose(l_input, (0, 2, 3, 4, 1)).reshape(H, W, K)
    dinv = jnp.asarray(_DINV)
    dinv2d = jnp.asarray(_DINV2D)

    out = pl.pallas_call(
        _body,
        grid=(NB,),
        in_specs=[
            pl.BlockSpec(memory_space=pltpu.MemorySpace.HBM),
            pl.BlockSpec(memory_space=pltpu.MemorySpace.HBM),
            pl.BlockSpec((HP, WP), lambda i: (0, 0)),
            pl.BlockSpec((K, C), lambda i: (0, 0)),
            pl.BlockSpec((1, C), lambda i: (0, 0)),
            pl.BlockSpec((C, C), lambda i: (0, 0)),
            pl.BlockSpec((1, C), lambda i: (0, 0)),
            pl.BlockSpec((C, C), lambda i: (0, 0)),
            pl.BlockSpec((1, C), lambda i: (0, 0)),
            pl.BlockSpec((C, 1), lambda i: (0, 0)),
            pl.BlockSpec((1, 1), lambda i: (0, 0)),
        ],
        out_specs=pl.BlockSpec((R, WP), lambda i: (i, 0)),
        out_shape=jax.ShapeDtypeStruct((H, WP), jnp.float32),
        scratch_shapes=[
            pltpu.VMEM((2, NR, WP, K), jnp.float32),
            pltpu.VMEM((2, P, C), jnp.float32),
            pltpu.SemaphoreType.DMA((2,)),
            pltpu.SemaphoreType.DMA((2,)),
        ],
    )(x3, dinv, dinv2d, W0, b0.reshape(1, C), Wr0, br0.reshape(1, C),
      Wr1, br1.reshape(1, C), W4, b4.reshape(1, 1))

    out = out[:, HALO:HALO + W]
    return out.reshape(1, 1, H, W, 1)


# speedup vs baseline: 148.2224x; 1.1687x over previous
import numpy as np
import jax
import jax.numpy as jnp
from jax.experimental import pallas as pl
from jax.experimental.pallas import tpu as pltpu

H = 224
W = 224
T = 16
F = 8
C = 64
K = T * F
HALO = 4
HP = H + 2 * HALO
WP = W + 2 * HALO
R = 56
NB = H // R
NR = R + 2 * HALO
P = NR * WP


def _dinv_const():
    vp = np.zeros(HP, np.float64)
    r = np.arange(HP) - HALO
    inside = (r >= 0) & (r < H)
    vp[inside] = 1 + (r[inside] > 0) + (r[inside] < H - 1)
    hp = np.zeros(WP, np.float64)
    c = np.arange(WP) - HALO
    insc = (c >= 0) & (c < W)
    hp[insc] = 1 + (c[insc] > 0) + (c[insc] < W - 1)
    deg = vp[:, None] * hp[None, :] + 1.0
    dinv = np.where(inside[:, None] & insc[None, :], 1.0 / np.sqrt(deg), 0.0)
    return dinv.astype(np.float32)


_DINV2D = _dinv_const()
_DINV = np.ascontiguousarray(
    np.broadcast_to(_DINV2D.reshape(HP * WP, 1), (HP * WP, C))
).astype(np.float32)


def _shift(a, k):
    return jnp.concatenate([a[-k:], a[:-k]], axis=0)


def _body(x_hbm, dv_hbm, dv2d, w0, b0, wr0, br0, wr1, br1, w4, b4, out_ref,
          xblk, dvblk, sem_x, sem_d):
    i = pl.program_id(0)
    slot = jax.lax.rem(i, 2)

    def xcopy(blk, s, start):
        def mk(src_r0, dst_r0, nrows):
            return pltpu.make_async_copy(
                x_hbm.at[pl.ds(src_r0, nrows), :, :],
                xblk.at[s, pl.ds(dst_r0, nrows), pl.ds(HALO, W), :],
                sem_x.at[s])

        @pl.when(blk == 0)
        def _():
            cp = mk(0, HALO, NR - HALO)
            cp.start() if start else cp.wait()

        r0 = jnp.maximum(blk * R - HALO, 0)

        @pl.when(jnp.logical_and(blk > 0, blk < NB - 1))
        def _():
            cp = mk(r0, 0, NR)
            cp.start() if start else cp.wait()

        @pl.when(blk == NB - 1)
        def _():
            cp = mk(r0, 0, NR - HALO)
            cp.start() if start else cp.wait()

    def dcopy(blk, s, start):
        cp = pltpu.make_async_copy(
            dv_hbm.at[pl.ds(blk * R * WP, P), :], dvblk.at[s], sem_d.at[s])
        cp.start() if start else cp.wait()

    @pl.when(i == 0)
    def _():
        xblk[0, :, 0:HALO, :] = jnp.zeros((NR, HALO, K), jnp.float32)
        xblk[1, :, 0:HALO, :] = jnp.zeros((NR, HALO, K), jnp.float32)
        xblk[0, :, HALO + W:WP, :] = jnp.zeros((NR, HALO, K), jnp.float32)
        xblk[1, :, HALO + W:WP, :] = jnp.zeros((NR, HALO, K), jnp.float32)
        xblk[0, 0:HALO, pl.ds(HALO, W), :] = jnp.zeros((HALO, W, K), jnp.float32)
        xcopy(0, 0, True)
        dcopy(0, 0, True)

    @pl.when(i + 1 < NB)
    def _():
        xcopy(i + 1, 1 - slot, True)
        dcopy(i + 1, 1 - slot, True)

    xcopy(i, slot, False)
    dcopy(i, slot, False)

    dinv = dvblk[slot]

    def agg(xw, brow):
        s = dinv * xw
        csum = s + _shift(s, 1) + _shift(s, -1)
        box = csum + _shift(csum, WP) + _shift(csum, -WP)
        return dinv * (box + s) + brow

    def gcn(inp, w, brow):
        return agg(jnp.dot(inp, w, preferred_element_type=jnp.float32), brow)

    x = jnp.reshape(xblk[slot], (P, K))
    h0 = gcn(x, w0[...], b0[...])
    h1 = gcn(jax.nn.relu(h0), wr0[...], br0[...])
    h2 = gcn(jax.nn.relu(h1), wr1[...], br1[...])
    ls = jax.nn.relu(h0 + h1 + h2)

    xw4 = jnp.dot(ls, w4[...], preferred_element_type=jnp.float32)
    f2 = xw4.reshape(NR, WP)
    dv2 = dv2d[pl.ds(i * R, NR), :]
    s3 = dv2 * f2
    c3 = s3 \
        + jnp.concatenate([s3[:, -1:], s3[:, :-1]], axis=1) \
        + jnp.concatenate([s3[:, 1:], s3[:, :1]], axis=1)
    b3 = c3 \
        + jnp.concatenate([c3[-1:], c3[:-1]], axis=0) \
        + jnp.concatenate([c3[1:], c3[:1]], axis=0)
    h4 = dv2 * (b3 + s3) + b4[...]
    out_ref[...] = h4[HALO:HALO + R, :]


def kernel(l_input, y, W0, b0, Wr0, br0, Wr1, br1, W4, b4):
    x3 = jnp.transpose(l_input, (0, 2, 3, 1, 4)).reshape(H, W, K)
    W0p = jnp.transpose(W0.reshape(F, T, C), (1, 0, 2)).reshape(K, C)
    dinv = jnp.asarray(_DINV)
    dinv2d = jnp.asarray(_DINV2D)

    out = pl.pallas_call(
        _body,
        grid=(NB,),
        in_specs=[
            pl.BlockSpec(memory_space=pltpu.MemorySpace.HBM),
            pl.BlockSpec(memory_space=pltpu.MemorySpace.HBM),
            pl.BlockSpec((HP, WP), lambda i: (0, 0)),
            pl.BlockSpec((K, C), lambda i: (0, 0)),
            pl.BlockSpec((1, C), lambda i: (0, 0)),
            pl.BlockSpec((C, C), lambda i: (0, 0)),
            pl.BlockSpec((1, C), lambda i: (0, 0)),
            pl.BlockSpec((C, C), lambda i: (0, 0)),
            pl.BlockSpec((1, C), lambda i: (0, 0)),
            pl.BlockSpec((C, 1), lambda i: (0, 0)),
            pl.BlockSpec((1, 1), lambda i: (0, 0)),
        ],
        out_specs=pl.BlockSpec((R, WP), lambda i: (i, 0)),
        out_shape=jax.ShapeDtypeStruct((H, WP), jnp.float32),
        scratch_shapes=[
            pltpu.VMEM((2, NR, WP, K), jnp.float32),
            pltpu.VMEM((2, P, C), jnp.float32),
            pltpu.SemaphoreType.DMA((2,)),
            pltpu.SemaphoreType.DMA((2,)),
        ],
    )(x3, dinv, dinv2d, W0p, b0.reshape(1, C), Wr0, br0.reshape(1, C),
      Wr1, br1.reshape(1, C), W4, b4.reshape(1, 1))

    out = out[:, HALO:HALO + W]
    return out.reshape(1, 1, H, W, 1)
